# Initial kernel scaffold; baseline (speedup 1.0000x reference)
import jax
import jax.numpy as jnp
from jax.experimental import pallas as pl


def kernel(encoder_indices, encoder_values, decoder_indices, ui_rows, ui_cols, ui_values, iu_values, img_feats, txt_feats, W_img, b_img, W_txt, b_txt, uEmb, iEmb, qTrans, kTrans, vTrans):
    raise NotImplementedError("write your pallas kernel here")


# speedup vs baseline: 2.0399x; 2.0399x over previous
import functools

import jax
import jax.numpy as jnp
from jax import lax
from jax.experimental import pallas as pl
from jax.experimental.pallas import tpu as pltpu
from jax.experimental.pallas import tpu_sc as plsc

USER = 5000
ITEM = 5000
N = USER + ITEM
D = 128
HEAD = 4
M_RATE = 0.5

NC = 2
NS = 16
NW = NC * NS
L = 16
K = 128

RN = 10112
RU = 5120


def _round_edges(e):
    return ((e + NW * K - 1) // (NW * K)) * (NW * K)


def _pad1(x, e_pad, fill):
    return jnp.pad(x, (0, e_pad - x.shape[0]), constant_values=fill)


def _zero_rows(buf, nrows, dt):
    def zrow(i, _):
        for j in range(dt // L):
            buf[i, pl.ds(j * L, L)] = jnp.zeros((L,), jnp.float32)
        return ()
    lax.fori_loop(0, nrows, zrow, ())


def _zero_acc_slice(acc, buf, nrows, sid, rows_per_sub):
    base_r = sid * rows_per_sub
    nz, rem = divmod(rows_per_sub, nrows)
    for t in range(nz):
        pltpu.sync_copy(buf, acc.at[pl.ds(base_r + t * nrows, nrows)])
    if rem:
        pltpu.sync_copy(buf.at[pl.ds(0, rem)],
                        acc.at[pl.ds(base_r + nz * nrows, rem)])


def _copy_out_slice(acc, buf, nrows, out_hbm, cid, sid, rows_per_sub):
    base_r = sid * rows_per_sub
    nz, rem = divmod(rows_per_sub, nrows)
    for t in range(nz):
        r0 = base_r + t * nrows
        pltpu.sync_copy(acc.at[pl.ds(r0, nrows)], buf)
        pltpu.sync_copy(buf, out_hbm.at[cid, pl.ds(r0, nrows)])
    if rem:
        r0 = base_r + nz * nrows
        pltpu.sync_copy(acc.at[pl.ds(r0, rem)], buf.at[pl.ds(0, rem)])
        pltpu.sync_copy(buf.at[pl.ds(0, rem)], out_hbm.at[cid, pl.ds(r0, rem)])


def _sc_spmm(src_idx, dst_idx, vals, table, r_out):
    e_pad = src_idx.shape[0]
    n_chunks = e_pad // (NW * K)
    dt = table.shape[1]
    rows_per_sub = r_out // NS
    mesh = plsc.VectorSubcoreMesh(core_axis_name="c", subcore_axis_name="s")

    @functools.partial(
        pl.kernel,
        out_type=jax.ShapeDtypeStruct((NC, r_out, dt), jnp.float32),
        mesh=mesh,
        compiler_params=pltpu.CompilerParams(needs_layout_passes=False),
        scratch_types=dict(
            acc=pltpu.VMEM_SHARED((r_out, dt), jnp.float32),
            src_i=pltpu.VMEM((K,), jnp.int32),
            dst_i=pltpu.VMEM((K,), jnp.int32),
            val_v=pltpu.VMEM((K + L,), jnp.float32),
            rows_v=pltpu.VMEM((K, dt), jnp.float32),
            sem=pltpu.SemaphoreType.DMA,
        ),
    )
    def spmm(src_hbm, dst_hbm, val_hbm, table_hbm, out_hbm,
             acc, src_i, dst_i, val_v, rows_v, sem):
        cid = lax.axis_index("c")
        sid = lax.axis_index("s")
        wid = sid * NC + cid
        _zero_rows(rows_v, K, dt)
        _zero_acc_slice(acc, rows_v, K, sid, rows_per_sub)
        plsc.subcore_barrier()

        e_per_w = n_chunks * K

        def chunk(ci, _):
            base = wid * e_per_w + ci * K
            pltpu.sync_copy(src_hbm.at[pl.ds(base, K)], src_i)
            pltpu.sync_copy(dst_hbm.at[pl.ds(base, K)], dst_i)
            pltpu.sync_copy(val_hbm.at[pl.ds(base, K)], val_v.at[pl.ds(0, K)])
            pltpu.async_copy(table_hbm.at[src_i], rows_v, sem).wait()

            def scale(e, _):
                v = val_v[pl.ds(e, L)][0]
                for j in range(dt // L):
                    rows_v[e, pl.ds(j * L, L)] = rows_v[e, pl.ds(j * L, L)] * v
                return ()
            lax.fori_loop(0, K, scale, ())
            pltpu.sync_copy(rows_v, acc.at[dst_i], add=True)
            return ()

        lax.fori_loop(0, n_chunks, chunk, ())
        plsc.subcore_barrier()
        _copy_out_slice(acc, rows_v, K, out_hbm, cid, sid, rows_per_sub)

    return spmm(src_idx, dst_idx, vals, table)


def _sc_attn_pass1(drows, dcols, q_tab, k_tab):
    e_pad = drows.shape[0]
    kc_ = K // 2
    n_chunks = e_pad // (NW * kc_)
    rows_per_sub = RN // NS
    mesh = plsc.VectorSubcoreMesh(core_axis_name="c", subcore_axis_name="s")

    @functools.partial(
        pl.kernel,
        out_type=(
            jax.ShapeDtypeStruct((e_pad, L), jnp.float32),
            jax.ShapeDtypeStruct((NC, RN, D), jnp.float32),
        ),
        mesh=mesh,
        compiler_params=pltpu.CompilerParams(needs_layout_passes=False),
        scratch_types=dict(
            acc=pltpu.VMEM_SHARED((RN, D), jnp.float32),
            ri=pltpu.VMEM((kc_,), jnp.int32),
            ci=pltpu.VMEM((kc_,), jnp.int32),
            qr=pltpu.VMEM((kc_, D), jnp.float32),
            kc=pltpu.VMEM((kc_, D), jnp.float32),
            abuf=pltpu.VMEM((kc_, L), jnp.float32),
            ab128=pltpu.VMEM((kc_, D), jnp.float32),
            sem=pltpu.SemaphoreType.DMA,
        ),
    )
    def pass1(drows_hbm, dcols_hbm, q_hbm, k_hbm, ea_hbm, norm_hbm,
              acc, ri, ci, qr, kc, abuf, ab128, sem):
        cid = lax.axis_index("c")
        sid = lax.axis_index("s")
        wid = sid * NC + cid
        _zero_rows(ab128, kc_, D)
        _zero_acc_slice(acc, ab128, kc_, sid, rows_per_sub)
        plsc.subcore_barrier()

        lane = lax.iota(jnp.int32, L)
        headmask = (lane < HEAD).astype(jnp.float32)
        e_per_w = n_chunks * kc_

        def chunk(ci_, _):
            base = wid * e_per_w + ci_ * kc_
            pltpu.sync_copy(drows_hbm.at[pl.ds(base, kc_)], ri)
            pltpu.sync_copy(dcols_hbm.at[pl.ds(base, kc_)], ci)
            pltpu.async_copy(q_hbm.at[ri], qr, sem).wait()
            pltpu.async_copy(k_hbm.at[ci], kc, sem).wait()

            def edge(e, _):
                row = jnp.zeros((L,), jnp.float32)
                for h in range(HEAD):
                    o = h * (D // HEAD)
                    ph = (qr[e, pl.ds(o, L)] * kc[e, pl.ds(o, L)]
                          + qr[e, pl.ds(o + L, L)] * kc[e, pl.ds(o + L, L)])
                    hs = jnp.sum(ph)
                    row = jnp.where(lane == h, hs, row)
                att = jnp.clip(row, -10.0, 10.0)
                ea = jnp.exp(att) * headmask
                abuf[e, :] = ea
                ab128[e, pl.ds(0, L)] = ea
                return ()
            lax.fori_loop(0, kc_, edge, ())
            pltpu.sync_copy(abuf, ea_hbm.at[pl.ds(base, kc_)])
            pltpu.sync_copy(ab128, acc.at[ri], add=True)
            return ()

        lax.fori_loop(0, n_chunks, chunk, ())
        plsc.subcore_barrier()
        _copy_out_slice(acc, qr, kc_, norm_hbm, cid, sid, rows_per_sub)

    return pass1(drows, dcols, q_tab, k_tab)


def _sc_attn_pass2(drows, dcols, v_tab, ea):
    e_pad = drows.shape[0]
    n_chunks = e_pad // (NW * K)
    rows_per_sub = RN // NS
    mesh = plsc.VectorSubcoreMesh(core_axis_name="c", subcore_axis_name="s")

    @functools.partial(
        pl.kernel,
        out_type=jax.ShapeDtypeStruct((NC, RN, D), jnp.float32),
        mesh=mesh,
        compiler_params=pltpu.CompilerParams(needs_layout_passes=False),
        scratch_types=dict(
            acc=pltpu.VMEM_SHARED((RN, D), jnp.float32),
            ri=pltpu.VMEM((K,), jnp.int32),
            ci=pltpu.VMEM((K,), jnp.int32),
            vc=pltpu.VMEM((K, D), jnp.float32),
            ab=pltpu.VMEM((K, L), jnp.float32),
            sem=pltpu.SemaphoreType.DMA,
        ),
    )
    def pass2(drows_hbm, dcols_hbm, v_hbm, ea_hbm, out_hbm,
              acc, ri, ci, vc, ab, sem):
        cid = lax.axis_index("c")
        sid = lax.axis_index("s")
        wid = sid * NC + cid
        _zero_rows(vc, K, D)
        _zero_acc_slice(acc, vc, K, sid, rows_per_sub)
        plsc.subcore_barrier()

        e_per_w = n_chunks * K

        def chunk(ci_, _):
            base = wid * e_per_w + ci_ * K
            pltpu.sync_copy(drows_hbm.at[pl.ds(base, K)], ri)
            pltpu.sync_copy(dcols_hbm.at[pl.ds(base, K)], ci)
            pltpu.sync_copy(ea_hbm.at[pl.ds(base, K)], ab)
            pltpu.async_copy(v_hbm.at[ci], vc, sem).wait()

            def edge(e, _):
                va = ab[e, :]
                for j in range(D // L):
                    a = va[j // 2]
                    vc[e, pl.ds(j * L, L)] = vc[e, pl.ds(j * L, L)] * a
                return ()
            lax.fori_loop(0, K, edge, ())
            pltpu.sync_copy(vc, acc.at[ri], add=True)
            return ()

        lax.fori_loop(0, n_chunks, chunk, ())
        plsc.subcore_barrier()
        _copy_out_slice(acc, vc, K, out_hbm, cid, sid, rows_per_sub)

    return pass2(drows, dcols, v_tab, ea)


_DN = (((1,), (1,)), ((), ()))


def _tc_linear(img_feats, w_img, b_img, txt_feats, w_txt, b_txt):
    def body(x1, w1, b1, x2, w2, b2, o1, o2):
        o1[...] = lax.dot_general(x1[...], w1[...], _DN,
                                  preferred_element_type=jnp.float32) + b1[...]
        o2[...] = lax.dot_general(x2[...], w2[...], _DN,
                                  preferred_element_type=jnp.float32) + b2[...]

    s = jax.ShapeDtypeStruct((ITEM, D), jnp.float32)
    return pl.pallas_call(
        body, out_shape=(s, s),
    )(img_feats, w_img, b_img.reshape(1, D), txt_feats, w_txt,
      b_txt.reshape(1, D))


def _tc_add2(parts):
    def body(p, o):
        o[...] = p[0] + p[1]

    r, c = parts.shape[1], parts.shape[2]
    return pl.pallas_call(
        body, out_shape=jax.ShapeDtypeStruct((r, c), jnp.float32))(parts)


def _tc_qkv(g2, q_w, k_w, v_w):
    def body(x, qw, kw, vw, qo, ko, vo):
        xv = x[...]
        qo[...] = jnp.dot(xv, qw[...], preferred_element_type=jnp.float32)
        ko[...] = jnp.dot(xv, kw[...], preferred_element_type=jnp.float32)
        vo[...] = jnp.dot(xv, vw[...], preferred_element_type=jnp.float32)

    s = jax.ShapeDtypeStruct((RN, D), jnp.float32)
    return pl.pallas_call(body, out_shape=(s, s, s))(g2, q_w, k_w, v_w)


def _tc_inv(norm_parts):
    def body(p, o):
        s = p[0] + p[1]
        cols = [jnp.broadcast_to(1.0 / (s[:, h:h + 1] + 1e-8), (RN, D // HEAD))
                for h in range(HEAD)]
        o[...] = jnp.concatenate(cols, axis=1)

    return pl.pallas_call(
        body, out_shape=jax.ShapeDtypeStruct((RN, D), jnp.float32))(norm_parts)


def _l2n(x):
    n = jnp.sqrt(jnp.sum(x * x, axis=1, keepdims=True))
    return x / jnp.maximum(n, 1e-12)


def _tc_combine(base, g1s, g2s, gt0, gt1, invs, img_parts, txt_parts):
    ni = len(img_parts)

    def body(b, a1, a2, a3, a4, iv, *rest):
        o = rest[-1]
        mi = rest[0][...]
        for extra in rest[1:ni]:
            mi = mi + extra[...]
        mt = rest[ni][...]
        for extra in rest[ni + 1:-1]:
            mt = mt + extra[...]
        s = b[...] + a1[...] + a2[...] + (a3[...] + a4[...]) * iv[...]
        o[...] = s + M_RATE * (_l2n(mi) + _l2n(mt))

    r = base.shape[0]
    return pl.pallas_call(
        body, out_shape=jax.ShapeDtypeStruct((r, D), jnp.float32))(
            base, g1s, g2s, gt0, gt1, invs, *img_parts, *txt_parts)


def kernel(encoder_indices, encoder_values, decoder_indices, ui_rows, ui_cols,
           ui_values, iu_values, img_feats, txt_feats, W_img, b_img, W_txt,
           b_txt, uEmb, iEmb, qTrans, kTrans, vTrans):
    e_ui, e_enc = ui_rows.shape[0], encoder_values.shape[0]
    e_ui_pad, e_enc_pad = _round_edges(e_ui), _round_edges(e_enc)

    ui_src = _pad1(ui_cols, e_ui_pad, 0)
    ui_dst = _pad1(ui_rows, e_ui_pad, 0)
    ui_val = _pad1(ui_values, e_ui_pad, 0.0)
    iu_val = _pad1(iu_values, e_ui_pad, 0.0)
    erows = _pad1(encoder_indices[0], e_enc_pad, 0)
    ecols = _pad1(encoder_indices[1], e_enc_pad, 0)
    evals = _pad1(encoder_values, e_enc_pad, 0.0)
    drows = _pad1(decoder_indices[0], e_enc_pad, N)
    dcols = _pad1(decoder_indices[1], e_enc_pad, 0)

    img, txt = _tc_linear(img_feats, W_img, b_img, txt_feats, W_txt, b_txt)

    img_u = _tc_add2(_sc_spmm(ui_src, ui_dst, ui_val, img, RU))
    txt_u = _tc_add2(_sc_spmm(ui_src, ui_dst, ui_val, txt, RU))
    img_i_parts = _sc_spmm(ui_dst, ui_src, iu_val, img_u, RU)
    txt_i_parts = _sc_spmm(ui_dst, ui_src, iu_val, txt_u, RU)

    cl = jnp.concatenate([uEmb, iEmb], axis=0)
    g1 = _tc_add2(_sc_spmm(ecols, erows, evals, cl, RN))
    g2 = _tc_add2(_sc_spmm(ecols, erows, evals, g1, RN))

    q_tab, k_tab, v_tab = _tc_qkv(g2, qTrans, kTrans, vTrans)

    ea, norm_parts = _sc_attn_pass1(drows, dcols, q_tab, k_tab)
    inv_tab = _tc_inv(norm_parts)
    gt_parts = _sc_attn_pass2(drows, dcols, v_tab, ea)

    u_embeds = _tc_combine(
        uEmb, g1[:USER], g2[:USER], gt_parts[0, :USER], gt_parts[1, :USER],
        inv_tab[:USER], (img_u[:USER],), (txt_u[:USER],))
    i_embeds = _tc_combine(
        iEmb, g1[USER:N], g2[USER:N], gt_parts[0, USER:N], gt_parts[1, USER:N],
        inv_tab[USER:N],
        (img_i_parts[0, :ITEM], img_i_parts[1, :ITEM]),
        (txt_i_parts[0, :ITEM], txt_i_parts[1, :ITEM]))
    return (u_embeds, i_embeds)
